# baseline (speedup 1.0000x reference)
import functools

import jax
import jax.numpy as jnp
from jax import lax
from jax.experimental import pallas as pl
from jax.experimental.pallas import tpu as pltpu
from jax.experimental.pallas import tpu_sc as plsc

NUM_NODES = 1000000
EMBED_DIM = 64
BATCH = 16384
N_FIELDS = 26
_IDS_PAD = 128

_NW = 32
_RPW = BATCH // _NW
_IPW = _RPW * N_FIELDS
_BROWS = 4
_CLEN = _BROWS * N_FIELDS
_NCH = _RPW // _BROWS
_NPAIR = _NCH // 2


def _make_kernel():
    mesh = plsc.VectorSubcoreMesh(core_axis_name="c", subcore_axis_name="s")

    @functools.partial(
        pl.kernel,
        mesh=mesh,
        compiler_params=pltpu.CompilerParams(use_tc_tiling_on_sc=False),
        out_type=jax.ShapeDtypeStruct((BATCH * N_FIELDS, EMBED_DIM),
                                      jnp.float32),
        scratch_types=[
            pltpu.VMEM((_RPW, 32), jnp.int32),
            pltpu.VMEM((_IPW + 8,), jnp.int32),
            pltpu.VMEM((2, _CLEN, EMBED_DIM), jnp.float32),
            pltpu.SemaphoreType.DMA,
            pltpu.SemaphoreType.DMA,
            pltpu.SemaphoreType.DMA,
            pltpu.SemaphoreType.DMA,
        ],
    )
    def gather_kernel(ids_hbm, table_hbm, out_hbm, idx32_v, idx_v, rows_v,
                      sem_ga, sem_gb, sem_oa, sem_ob):
        wid = lax.axis_index("s") * 2 + lax.axis_index("c")
        base = wid * _RPW
        obase = wid * _IPW
        pltpu.sync_copy(ids_hbm.at[pl.ds(base, _RPW), pl.ds(0, 32)], idx32_v)

        def compact(r, carry):
            idx_v[pl.ds(r * N_FIELDS, 16)] = idx32_v[r, pl.ds(0, 16)]
            idx_v[pl.ds(r * N_FIELDS + 16, 16)] = idx32_v[r, pl.ds(16, 16)]
            return carry

        lax.fori_loop(0, _RPW, compact, 0)

        def start_gather(c, half, sem):
            pltpu.async_copy(
                table_hbm.at[idx_v.at[pl.ds(c * _CLEN, _CLEN)]],
                rows_v.at[half], sem)

        def wait_gather(half, sem):
            pltpu.make_async_copy(
                table_hbm.at[idx_v.at[pl.ds(0, _CLEN)]],
                rows_v.at[half], sem).wait()

        def start_out(c, half, sem):
            pltpu.async_copy(
                rows_v.at[half],
                out_hbm.at[pl.ds(obase + c * _CLEN, _CLEN)], sem)

        def wait_out(c, half, sem):
            pltpu.make_async_copy(
                rows_v.at[half],
                out_hbm.at[pl.ds(obase + c * _CLEN, _CLEN)], sem).wait()

        start_gather(0, 0, sem_ga)

        def body(k, carry):
            c0 = 2 * k
            c1 = 2 * k + 1
            wait_gather(0, sem_ga)

            @pl.when(k > 0)
            def _():
                wait_out(c1 - 2, 1, sem_ob)

            start_gather(c1, 1, sem_gb)
            start_out(c0, 0, sem_oa)
            wait_gather(1, sem_gb)

            @pl.when(k < _NPAIR - 1)
            def _():
                wait_out(c0, 0, sem_oa)
                start_gather(c0 + 2, 0, sem_ga)

            start_out(c1, 1, sem_ob)
            return carry

        lax.fori_loop(0, _NPAIR, body, 0)

        wait_out(_NCH - 2, 0, sem_oa)
        wait_out(_NCH - 1, 1, sem_ob)

    return gather_kernel


_gather = _make_kernel()


def kernel(ids, weight):
    ids32 = ids.astype(jnp.int32)
    ids_p = jnp.pad(ids32, ((0, 0), (0, _IDS_PAD - N_FIELDS)))
    out = _gather(ids_p, weight)
    return out.reshape(BATCH, N_FIELDS, EMBED_DIM)


# speedup vs baseline: 1.0372x; 1.0372x over previous
import functools

import jax
import jax.numpy as jnp
from jax import lax
from jax.experimental import pallas as pl
from jax.experimental.pallas import tpu as pltpu
from jax.experimental.pallas import tpu_sc as plsc

NUM_NODES = 1000000
EMBED_DIM = 64
BATCH = 16384
N_FIELDS = 26
_IDS_PAD = 128
_PADW = 128

_NW = 32
_RPW = BATCH // _NW
_IPW = _RPW * N_FIELDS
_CLEN = 128
_NCH = _IPW // _CLEN
_NPAIR = _NCH // 2
_TOTAL = BATCH * N_FIELDS


def _make_kernel():
    mesh = plsc.VectorSubcoreMesh(core_axis_name="c", subcore_axis_name="s")

    @functools.partial(
        pl.kernel,
        mesh=mesh,
        compiler_params=pltpu.CompilerParams(use_tc_tiling_on_sc=False),
        out_type=jax.ShapeDtypeStruct((_TOTAL, EMBED_DIM), jnp.float32),
        scratch_types=[
            pltpu.VMEM((_RPW, 32), jnp.int32),
            pltpu.VMEM((_IPW + 8,), jnp.int32),
            pltpu.VMEM((2, _CLEN, _PADW), jnp.float32),
            pltpu.SemaphoreType.DMA,
            pltpu.SemaphoreType.DMA,
            pltpu.SemaphoreType.DMA,
            pltpu.SemaphoreType.DMA,
        ],
    )
    def gather_kernel(ids_hbm, table_hbm, out_hbm, idx32_v, idx_v, rows_v,
                      sem_ga, sem_gb, sem_oa, sem_ob):
        wid = lax.axis_index("s") * 2 + lax.axis_index("c")
        base = wid * _RPW
        obase = wid * _IPW

        pltpu.sync_copy(ids_hbm.at[pl.ds(base, _RPW), pl.ds(0, 32)], idx32_v)

        def compact(r, carry):
            idx_v[pl.ds(r * N_FIELDS, 16)] = idx32_v[r, pl.ds(0, 16)]
            idx_v[pl.ds(r * N_FIELDS + 16, 16)] = idx32_v[r, pl.ds(16, 16)]
            return carry

        lax.fori_loop(0, _RPW, compact, 0)

        def start_gather(c, half, sem):
            pltpu.async_copy(
                table_hbm.at[idx_v.at[pl.ds(c * _CLEN, _CLEN)]],
                rows_v.at[half], sem)

        def wait_gather(half, sem):
            pltpu.make_async_copy(
                table_hbm.at[idx_v.at[pl.ds(0, _CLEN)]],
                rows_v.at[half], sem).wait()

        def start_out(c, half, sem):
            pltpu.async_copy(
                rows_v.at[half].at[pl.ds(0, _CLEN), pl.ds(0, EMBED_DIM)],
                out_hbm.at[pl.ds(obase + c * _CLEN, _CLEN)], sem)

        def wait_out(c, half, sem):
            pltpu.make_async_copy(
                rows_v.at[half].at[pl.ds(0, _CLEN), pl.ds(0, EMBED_DIM)],
                out_hbm.at[pl.ds(obase + c * _CLEN, _CLEN)], sem).wait()

        start_gather(0, 0, sem_ga)

        def body(k, carry):
            c0 = 2 * k
            c1 = 2 * k + 1
            wait_gather(0, sem_ga)

            @pl.when(k > 0)
            def _():
                wait_out(c1 - 2, 1, sem_ob)

            start_gather(c1, 1, sem_gb)
            start_out(c0, 0, sem_oa)
            wait_gather(1, sem_gb)

            @pl.when(k < _NPAIR - 1)
            def _():
                wait_out(c0, 0, sem_oa)
                start_gather(c0 + 2, 0, sem_ga)

            start_out(c1, 1, sem_ob)
            return carry

        lax.fori_loop(0, _NPAIR, body, 0)

        wait_out(_NCH - 2, 0, sem_oa)
        wait_out(_NCH - 1, 1, sem_ob)

    return gather_kernel


_gather = _make_kernel()


def kernel(ids, weight):
    ids32 = ids.astype(jnp.int32)
    ids_p = jnp.pad(ids32, ((0, 0), (0, _IDS_PAD - N_FIELDS)))
    weight_p = jnp.pad(weight, ((0, 0), (0, _PADW - EMBED_DIM)))
    out = _gather(ids_p, weight_p)
    return out.reshape(BATCH, N_FIELDS, EMBED_DIM)


# speedup vs baseline: 1.0932x; 1.0540x over previous
import functools

import jax
import jax.numpy as jnp
from jax import lax
from jax.experimental import pallas as pl
from jax.experimental.pallas import tpu as pltpu
from jax.experimental.pallas import tpu_sc as plsc

NUM_NODES = 1000000
EMBED_DIM = 64
BATCH = 16384
N_FIELDS = 26
_IDS_PAD = 128
_PADW = 128

_NW = 32
_RPW = BATCH // _NW
_IPW = _RPW * N_FIELDS
_CLEN = 128
_NCH = _IPW // _CLEN
_NBUF = 4
_NGRP = _NCH // _NBUF
_TOTAL = BATCH * N_FIELDS


def _make_kernel():
    mesh = plsc.VectorSubcoreMesh(core_axis_name="c", subcore_axis_name="s")

    @functools.partial(
        pl.kernel,
        mesh=mesh,
        compiler_params=pltpu.CompilerParams(use_tc_tiling_on_sc=False),
        out_type=jax.ShapeDtypeStruct((_TOTAL, EMBED_DIM), jnp.float32),
        scratch_types=[
            pltpu.VMEM((_RPW, 32), jnp.int32),
            pltpu.VMEM((_IPW + 8,), jnp.int32),
            pltpu.VMEM((_NBUF, _CLEN, _PADW), jnp.float32),
            pltpu.SemaphoreType.DMA,
            pltpu.SemaphoreType.DMA,
            pltpu.SemaphoreType.DMA,
            pltpu.SemaphoreType.DMA,
            pltpu.SemaphoreType.DMA,
            pltpu.SemaphoreType.DMA,
            pltpu.SemaphoreType.DMA,
            pltpu.SemaphoreType.DMA,
        ],
    )
    def gather_kernel(ids_hbm, table_hbm, out_hbm, idx32_v, idx_v, rows_v,
                      sg0, sg1, sg2, sg3, so0, so1, so2, so3):
        sem_g = [sg0, sg1, sg2, sg3]
        sem_o = [so0, so1, so2, so3]
        wid = lax.axis_index("s") * 2 + lax.axis_index("c")
        base = wid * _RPW
        obase = wid * _IPW

        pltpu.sync_copy(ids_hbm.at[pl.ds(base, _RPW), pl.ds(0, 32)], idx32_v)

        def compact(r, carry):
            idx_v[pl.ds(r * N_FIELDS, 16)] = idx32_v[r, pl.ds(0, 16)]
            idx_v[pl.ds(r * N_FIELDS + 16, 16)] = idx32_v[r, pl.ds(16, 16)]
            return carry

        lax.fori_loop(0, _RPW, compact, 0)

        def start_gather(c, b):
            pltpu.async_copy(
                table_hbm.at[idx_v.at[pl.ds(c * _CLEN, _CLEN)]],
                rows_v.at[b], sem_g[b])

        def wait_gather(b):
            pltpu.make_async_copy(
                table_hbm.at[idx_v.at[pl.ds(0, _CLEN)]],
                rows_v.at[b], sem_g[b]).wait()

        def start_out(c, b):
            pltpu.async_copy(
                rows_v.at[b].at[pl.ds(0, _CLEN), pl.ds(0, EMBED_DIM)],
                out_hbm.at[pl.ds(obase + c * _CLEN, _CLEN)], sem_o[b])

        def wait_out(c, b):
            pltpu.make_async_copy(
                rows_v.at[b].at[pl.ds(0, _CLEN), pl.ds(0, EMBED_DIM)],
                out_hbm.at[pl.ds(obase + c * _CLEN, _CLEN)], sem_o[b]).wait()

        for b in range(_NBUF):
            start_gather(b, b)

        def body(k, carry):
            for b in range(_NBUF):
                c = _NBUF * k + b
                wait_gather(b)
                start_out(c, b)

                @pl.when(k < _NGRP - 1)
                def _():
                    wait_out(c, b)
                    start_gather(c + _NBUF, b)

            return carry

        lax.fori_loop(0, _NGRP, body, 0)

        for b in range(_NBUF):
            wait_out(_NCH - _NBUF + b, b)

    return gather_kernel


_gather = _make_kernel()


def kernel(ids, weight):
    ids32 = ids.astype(jnp.int32)
    ids_p = jnp.pad(ids32, ((0, 0), (0, _IDS_PAD - N_FIELDS)))
    weight_p = jnp.pad(weight, ((0, 0), (0, _PADW - EMBED_DIM)))
    out = _gather(ids_p, weight_p)
    return out.reshape(BATCH, N_FIELDS, EMBED_DIM)


# speedup vs baseline: 1.2411x; 1.1353x over previous
import functools

import jax
import jax.numpy as jnp
from jax import lax
from jax.experimental import pallas as pl
from jax.experimental.pallas import tpu as pltpu
from jax.experimental.pallas import tpu_sc as plsc

NUM_NODES = 1000000
EMBED_DIM = 64
BATCH = 16384
N_FIELDS = 26
_FPAD = 32
_IDS_PAD = 128
_PADW = 128

_NW = 32
_RPW = BATCH // _NW
_IPW = _RPW * _FPAD
_CLEN = 128
_NCH = _IPW // _CLEN
_NBUF = 4
_NGRP = _NCH // _NBUF


def _make_kernel():
    mesh = plsc.VectorSubcoreMesh(core_axis_name="c", subcore_axis_name="s")

    @functools.partial(
        pl.kernel,
        mesh=mesh,
        compiler_params=pltpu.CompilerParams(use_tc_tiling_on_sc=False),
        out_type=jax.ShapeDtypeStruct((BATCH * _FPAD, _PADW), jnp.float32),
        scratch_types=[
            pltpu.VMEM((_RPW, _FPAD), jnp.int32),
            pltpu.VMEM((_IPW,), jnp.int32),
            pltpu.VMEM((_NBUF, _CLEN, _PADW), jnp.float32),
            pltpu.SemaphoreType.DMA,
            pltpu.SemaphoreType.DMA,
            pltpu.SemaphoreType.DMA,
            pltpu.SemaphoreType.DMA,
            pltpu.SemaphoreType.DMA,
            pltpu.SemaphoreType.DMA,
            pltpu.SemaphoreType.DMA,
            pltpu.SemaphoreType.DMA,
        ],
    )
    def gather_kernel(ids_hbm, table_hbm, out_hbm, idx32_v, idx_v, rows_v,
                      sg0, sg1, sg2, sg3, so0, so1, so2, so3):
        sem_g = [sg0, sg1, sg2, sg3]
        sem_o = [so0, so1, so2, so3]
        wid = lax.axis_index("s") * 2 + lax.axis_index("c")
        base = wid * _RPW
        obase = wid * _IPW

        pltpu.sync_copy(
            ids_hbm.at[pl.ds(base, _RPW), pl.ds(0, _FPAD)], idx32_v)

        def flatten(r, carry):
            idx_v[pl.ds(r * _FPAD, 16)] = idx32_v[r, pl.ds(0, 16)]
            idx_v[pl.ds(r * _FPAD + 16, 16)] = idx32_v[r, pl.ds(16, 16)]
            return carry

        lax.fori_loop(0, _RPW, flatten, 0)

        def start_gather(c, b):
            pltpu.async_copy(
                table_hbm.at[idx_v.at[pl.ds(c * _CLEN, _CLEN)]],
                rows_v.at[b], sem_g[b])

        def wait_gather(b):
            pltpu.make_async_copy(
                table_hbm.at[idx_v.at[pl.ds(0, _CLEN)]],
                rows_v.at[b], sem_g[b]).wait()

        def start_out(c, b):
            pltpu.async_copy(
                rows_v.at[b],
                out_hbm.at[pl.ds(obase + c * _CLEN, _CLEN)], sem_o[b])

        def wait_out(c, b):
            pltpu.make_async_copy(
                rows_v.at[b],
                out_hbm.at[pl.ds(obase + c * _CLEN, _CLEN)], sem_o[b]).wait()

        for b in range(_NBUF):
            start_gather(b, b)

        def body(k, carry):
            for b in range(_NBUF):
                c = _NBUF * k + b
                wait_gather(b)
                start_out(c, b)

                @pl.when(k < _NGRP - 1)
                def _():
                    wait_out(c, b)
                    start_gather(c + _NBUF, b)

            return carry

        lax.fori_loop(0, _NGRP, body, 0)

        for b in range(_NBUF):
            wait_out(_NCH - _NBUF + b, b)

    return gather_kernel


_gather = _make_kernel()


def kernel(ids, weight):
    ids32 = ids.astype(jnp.int32)
    dummy = (
        lax.broadcasted_iota(jnp.int32, (BATCH, _FPAD - N_FIELDS), 0) * 7
        + lax.broadcasted_iota(jnp.int32, (BATCH, _FPAD - N_FIELDS), 1)
    ) % NUM_NODES
    ids_p = jnp.pad(
        jnp.concatenate([ids32, dummy], axis=1),
        ((0, 0), (0, _IDS_PAD - _FPAD)))
    weight_p = jnp.pad(weight, ((0, 0), (0, _PADW - EMBED_DIM)))
    out = _gather(ids_p, weight_p)
    return out.reshape(BATCH, _FPAD, _PADW)[:, :N_FIELDS, :EMBED_DIM]
